# baseline (speedup 1.0000x reference)
import functools

import jax
import jax.numpy as jnp
from jax import lax
from jax.experimental import pallas as pl
from jax.experimental.pallas import tpu as pltpu
from jax.experimental.pallas import tpu_sc as plsc

N_NODES = 10000
D = 128
NC = 2
NS = 16
NW = NC * NS
CHUNK = 128
NCH = 81
NBUF = 3
E_PAD = NW * NCH * CHUNK
N_PAD = 10112
RPT = N_PAD // NS
N_PAD_DEG = 10240
RPT_DEG = N_PAD_DEG // NS


def _sc_mesh():
    return plsc.VectorSubcoreMesh(
        core_axis_name="c", subcore_axis_name="s", num_cores=NC, num_subcores=NS
    )


def _deg_body(edges_hbm, zrow_hbm, deg_out, idx_v, ones_v, sem_i, deg_sp):
    c = lax.axis_index("c")
    s = lax.axis_index("s")
    w = c * NS + s
    pltpu.sync_copy(zrow_hbm, deg_sp.at[pl.ds(s * RPT_DEG, RPT_DEG)])
    for i in range(CHUNK // 16):
        ones_v[pl.ds(i * 16, 16)] = jnp.ones((16,), jnp.float32)

    def start_idx(j, b):
        pltpu.async_copy(edges_hbm.at[w, j], idx_v.at[b], sem_i.at[b])

    def wait_idx(j, b):
        pltpu.make_async_copy(edges_hbm.at[w, j], idx_v.at[b],
                              sem_i.at[b]).wait()

    for b in range(NBUF - 1):
        start_idx(b, b)
    plsc.subcore_barrier()

    @pl.loop(0, NCH, step=NBUF)
    def _(g):
        for b in range(NBUF):
            j = g + b
            wait_idx(j, b)
            pltpu.sync_copy(ones_v, deg_sp.at[idx_v.at[b, 1]], add=True)

            @pl.when(j + NBUF - 1 < NCH)
            def _():
                start_idx(j + NBUF - 1, (b - 1) % NBUF)

    plsc.subcore_barrier()
    pltpu.sync_copy(deg_sp.at[pl.ds(s * RPT_DEG, RPT_DEG)], deg_out.at[c, s])


def _degree_pass(edges_w):
    zrow = jnp.zeros((RPT_DEG,), jnp.float32)
    k = pl.kernel(
        _deg_body,
        out_type=jax.ShapeDtypeStruct((NC, NS, RPT_DEG), jnp.float32),
        mesh=_sc_mesh(),
        scratch_types=[
            pltpu.VMEM((NBUF, 2, CHUNK), jnp.int32),
            pltpu.VMEM((CHUNK,), jnp.float32),
            pltpu.SemaphoreType.DMA((NBUF,)),
            pltpu.VMEM_SHARED((N_PAD_DEG,), jnp.float32),
        ],
    )
    return k(edges_w, zrow)


def _scat_body(y_hbm, edges_hbm, ztile_hbm, acc_out,
               idx_v, rows_v, sem_i, sem_g, sem_s, acc_sp):
    c = lax.axis_index("c")
    s = lax.axis_index("s")
    w = c * NS + s
    pltpu.sync_copy(ztile_hbm, acc_sp.at[pl.ds(s * RPT, RPT)])

    def start_idx(j, b):
        pltpu.async_copy(edges_hbm.at[w, j], idx_v.at[b], sem_i.at[b])

    def wait_idx(j, b):
        pltpu.make_async_copy(edges_hbm.at[w, j], idx_v.at[b],
                              sem_i.at[b]).wait()

    def start_gather(b):
        pltpu.async_copy(y_hbm.at[idx_v.at[b, 0]], rows_v.at[b], sem_g.at[b])

    def wait_gather(b):
        pltpu.make_async_copy(y_hbm.at[idx_v.at[b, 0]], rows_v.at[b],
                              sem_g.at[b]).wait()

    def start_scatter(b):
        pltpu.async_copy(rows_v.at[b], acc_sp.at[idx_v.at[b, 1]], sem_s.at[b],
                         add=True)

    def wait_scatter(b):
        pltpu.make_async_copy(rows_v.at[b], acc_sp.at[idx_v.at[b, 1]],
                              sem_s.at[b]).wait()

    for b in range(NBUF - 1):
        start_idx(b, b)
    for b in range(NBUF - 1):
        wait_idx(b, b)
        start_gather(b)
    plsc.subcore_barrier()

    @pl.loop(0, NCH, step=NBUF)
    def _(g):
        for b in range(NBUF):
            j = g + b
            wait_gather(b)
            start_scatter(b)

            @pl.when(j >= 1)
            def _():
                wait_scatter((b - 1) % NBUF)

            @pl.when(j + NBUF - 1 < NCH)
            def _():
                start_idx(j + NBUF - 1, (b - 1) % NBUF)
                wait_idx(j + NBUF - 1, (b - 1) % NBUF)
                start_gather((b - 1) % NBUF)

    wait_scatter((NCH - 1) % NBUF)
    plsc.subcore_barrier()
    pltpu.sync_copy(acc_sp.at[pl.ds(s * RPT, RPT)], acc_out.at[c, s])


def _scatter_pass(y, edges_w):
    ztile = jnp.zeros((RPT, D), jnp.float32)
    k = pl.kernel(
        _scat_body,
        out_type=jax.ShapeDtypeStruct((NC, NS, RPT, D), jnp.float32),
        mesh=_sc_mesh(),
        scratch_types=[
            pltpu.VMEM((NBUF, 2, CHUNK), jnp.int32),
            pltpu.VMEM((NBUF, CHUNK, D), jnp.float32),
            pltpu.SemaphoreType.DMA((NBUF,)),
            pltpu.SemaphoreType.DMA((NBUF,)),
            pltpu.SemaphoreType.DMA((NBUF,)),
            pltpu.VMEM_SHARED((N_PAD, D), jnp.float32),
        ],
    )
    return k(y, edges_w, ztile)


BLK = 1264


def _mm1_body(x_ref, w_ref, d0_ref, d1_ref, y_ref, dinv_ref):
    dinv = lax.rsqrt(d0_ref[...] + d1_ref[...] + 1.0)
    y_ref[...] = dinv * jnp.dot(x_ref[...], w_ref[...],
                                preferred_element_type=jnp.float32)
    dinv_ref[...] = dinv


def _tc_layer1(x, W1, deg_parts):
    d0 = deg_parts[0].reshape(N_PAD, 1)
    d1 = deg_parts[1].reshape(N_PAD, 1)
    grid = (N_PAD // BLK,)
    return pl.pallas_call(
        _mm1_body,
        grid=grid,
        in_specs=[
            pl.BlockSpec((BLK, D), lambda i: (i, 0)),
            pl.BlockSpec((D, D), lambda i: (0, 0)),
            pl.BlockSpec((BLK, 1), lambda i: (i, 0)),
            pl.BlockSpec((BLK, 1), lambda i: (i, 0)),
        ],
        out_specs=[
            pl.BlockSpec((BLK, D), lambda i: (i, 0)),
            pl.BlockSpec((BLK, 1), lambda i: (i, 0)),
        ],
        out_shape=[
            jax.ShapeDtypeStruct((N_PAD, D), jnp.float32),
            jax.ShapeDtypeStruct((N_PAD, 1), jnp.float32),
        ],
    )(x, W1, d0, d1)


def _mid_body(a0_ref, a1_ref, y1_ref, dinv_ref, b_ref, w_ref, y2_ref):
    dinv = dinv_ref[...]
    h = dinv * (a0_ref[...] + a1_ref[...] + y1_ref[...]) + b_ref[...]
    h = jnp.maximum(h, 0.0)
    y2_ref[...] = dinv * jnp.dot(h, w_ref[...],
                                 preferred_element_type=jnp.float32)


def _tc_mid(a0, a1, y1, dinv, b1, W2):
    grid = (N_PAD // BLK,)
    return pl.pallas_call(
        _mid_body,
        grid=grid,
        in_specs=[
            pl.BlockSpec((BLK, D), lambda i: (i, 0)),
            pl.BlockSpec((BLK, D), lambda i: (i, 0)),
            pl.BlockSpec((BLK, D), lambda i: (i, 0)),
            pl.BlockSpec((BLK, 1), lambda i: (i, 0)),
            pl.BlockSpec((1, D), lambda i: (0, 0)),
            pl.BlockSpec((D, D), lambda i: (0, 0)),
        ],
        out_specs=pl.BlockSpec((BLK, D), lambda i: (i, 0)),
        out_shape=jax.ShapeDtypeStruct((N_PAD, D), jnp.float32),
    )(a0, a1, y1, dinv, b1, W2)


def _fin_body(a0_ref, a1_ref, y2_ref, dinv_ref, b_ref, z_ref):
    z_ref[...] = (dinv_ref[...] * (a0_ref[...] + a1_ref[...] + y2_ref[...])
                  + b_ref[...])


def _tc_final(a0, a1, y2, dinv, b2):
    grid = (N_PAD // BLK,)
    return pl.pallas_call(
        _fin_body,
        grid=grid,
        in_specs=[
            pl.BlockSpec((BLK, D), lambda i: (i, 0)),
            pl.BlockSpec((BLK, D), lambda i: (i, 0)),
            pl.BlockSpec((BLK, D), lambda i: (i, 0)),
            pl.BlockSpec((BLK, 1), lambda i: (i, 0)),
            pl.BlockSpec((1, D), lambda i: (0, 0)),
        ],
        out_specs=pl.BlockSpec((BLK, D), lambda i: (i, 0)),
        out_shape=jax.ShapeDtypeStruct((N_PAD, D), jnp.float32),
    )(a0, a1, y2, dinv, b2)


def kernel(x, edge_index, W1, b1, W2, b2):
    x_pad = jnp.pad(x, ((0, N_PAD - N_NODES), (0, 0)))
    pad_e = E_PAD - edge_index.shape[1]
    edges_w = jnp.pad(edge_index, ((0, 0), (0, pad_e)),
                      constant_values=N_NODES)
    edges_w = edges_w.reshape(2, NW, NCH, CHUNK).transpose(1, 2, 0, 3)
    b1r = b1.reshape(1, D)
    b2r = b2.reshape(1, D)

    deg_parts = _degree_pass(edges_w).reshape(NC, N_PAD_DEG)[:, :N_PAD]
    y1, dinv = _tc_layer1(x_pad, W1, deg_parts)
    acc1 = _scatter_pass(y1, edges_w).reshape(NC, N_PAD, D)
    y2 = _tc_mid(acc1[0], acc1[1], y1, dinv, b1r, W2)
    acc2 = _scatter_pass(y2, edges_w).reshape(NC, N_PAD, D)
    z = _tc_final(acc2[0], acc2[1], y2, dinv, b2r)
    return z[:N_NODES]
